# baseline (speedup 1.0000x reference)
import functools

import jax
import jax.numpy as jnp
from jax.experimental import pallas as pl
from jax.experimental.pallas import tpu as pltpu

N = 10000
E = 160000
B = 64
H = 2


def _mm_body(x_ref, w_ref, b_ref, o_ref, *, act):
    acc = jnp.dot(x_ref[...], w_ref[...], preferred_element_type=jnp.float32)
    acc = acc + b_ref[...]
    if act == "relu":
        acc = jnp.maximum(acc, 0.0)
    elif act == "tanh":
        acc = jnp.tanh(acc)
    o_ref[...] = acc


@functools.partial(jax.jit, static_argnames=("act", "block_m"))
def matmul(x, w, b=None, act="none", block_m=512):
    m, k = x.shape
    k2, n = w.shape
    assert k == k2
    if b is None:
        b = jnp.zeros((n,), jnp.float32)
    b = b.reshape(1, n)
    nb = pl.cdiv(m, block_m)
    return pl.pallas_call(
        functools.partial(_mm_body, act=act),
        grid=(nb,),
        in_specs=[
            pl.BlockSpec((block_m, k), lambda i: (i, 0)),
            pl.BlockSpec((k, n), lambda i: (0, 0)),
            pl.BlockSpec((1, n), lambda i: (0, 0)),
        ],
        out_specs=pl.BlockSpec((block_m, n), lambda i: (i, 0)),
        out_shape=jax.ShapeDtypeStruct((m, n), jnp.float32),
    )(x, w, b)


def _loops(edge_index, n):
    ar = jnp.arange(n, dtype=edge_index.dtype)
    src = jnp.concatenate([edge_index[0], ar])
    dst = jnp.concatenate([edge_index[1], ar])
    return src, dst


def _gat(x, src, dst, prm, n, act):
    fout = prm["b"].shape[0]
    h = matmul(x, prm["W"]).reshape(n, H, fout)
    a = jnp.sum(h * prm["asrc"][None], axis=-1)[src] + jnp.sum(h * prm["adst"][None], axis=-1)[dst]
    a = jax.nn.leaky_relu(a, negative_slope=0.2)
    m = jax.ops.segment_max(a, dst, num_segments=n)
    m = jnp.where(jnp.isfinite(m), m, 0.0)
    ex = jnp.exp(a - m[dst])
    den = jax.ops.segment_sum(ex, dst, num_segments=n)
    att = ex / (den[dst] + 1e-16)
    out = jax.ops.segment_sum(h[src] * att[:, :, None], dst, num_segments=n)
    out = out.mean(axis=1) + prm["b"]
    if act == "relu":
        out = jnp.maximum(out, 0.0)
    return out


def _gcn(x, src, dst, w, prm, n):
    deg = jax.ops.segment_sum(w, dst, num_segments=n)
    dinv = jnp.where(deg > 0, deg ** -0.5, 0.0)
    norm = dinv[src] * w * dinv[dst]
    h = matmul(x, prm["W"])
    out = jax.ops.segment_sum(h[src] * norm[:, None], dst, num_segments=n) + prm["b"]
    return jnp.maximum(out, 0.0)


def _gmp(x, batch, b):
    out = jax.ops.segment_max(x, batch, num_segments=b)
    return jnp.where(jnp.isfinite(out), out, 0.0)


def _head_body(x_ref, xt_ref, xq_ref, *refs):
    (mf1w, mf1b, mf2w, mf2b, pf1w, pf1b, pf2w, pf2b, cf1w, cf1b, cf2w, cf2b,
     ax1, axb, ax2, at1, atb, at2, aq1, aqb, aq2,
     f1w, f1b, f2w, f2b, ow, ob, o_ref) = refs
    dot = lambda a, b: jnp.dot(a, b, preferred_element_type=jnp.float32)
    relu = lambda v: jnp.maximum(v, 0.0)
    x = dot(relu(dot(x_ref[...], mf1w[...]) + mf1b[...]), mf2w[...]) + mf2b[...]
    xt = dot(relu(dot(xt_ref[...], pf1w[...]) + pf1b[...]), pf2w[...]) + pf2b[...]
    xq = dot(relu(dot(xq_ref[...], cf1w[...]) + cf1b[...]), cf2w[...]) + cf2b[...]
    ax = dot(jnp.tanh(dot(x, ax1[...]) + axb[...]), ax2[...])
    at = dot(jnp.tanh(dot(xt, at1[...]) + atb[...]), at2[...])
    aq = dot(jnp.tanh(dot(xq, aq1[...]) + aqb[...]), aq2[...])
    a = jnp.concatenate([ax, at, aq], axis=1)
    a = jax.nn.softmax(a, axis=1)
    emb = jnp.concatenate(
        [a[:, 0:1] * x, a[:, 1:2] * xt, a[:, 2:3] * xq], axis=1)
    xc = relu(dot(emb, f1w[...]) + f1b[...])
    xc = relu(dot(xc, f2w[...]) + f2b[...])
    o_ref[...] = dot(xc, ow[...]) + ob[...]


def _head(x, xt, xq, params):
    args = [x, xt, xq]
    for nm in ("mol_fc1", "mol_fc2", "pro_fc1", "pro_fc2", "clq_fc1", "clq_fc2"):
        args += [params[nm]["W"], params[nm]["b"].reshape(1, -1)]
    for nm in ("att_x", "att_xt", "att_xq"):
        args += [params[nm]["W1"], params[nm]["b1"].reshape(1, -1), params[nm]["W2"]]
    for nm in ("fc1", "fc2", "out"):
        args += [params[nm]["W"], params[nm]["b"].reshape(1, -1)]
    return pl.pallas_call(
        _head_body,
        out_shape=jax.ShapeDtypeStruct((B, 1), jnp.float32),
    )(*args)


def kernel(mol_x, pro_x, clique_x, pro_edge_weight, params, mol_edge_index,
           pro_edge_index, clique_edge_index, mol_batch, pro_batch, clique_batch):
    ms, md = _loops(mol_edge_index, N)
    h = _gat(mol_x, ms, md, params["mol0"], N, "relu")
    h = _gat(h, ms, md, params["mol1"], N, "relu")
    h = _gat(h, ms, md, params["mol2"], N, "none")
    x = _gmp(h, mol_batch, B)

    ps, pd = _loops(pro_edge_index, N)
    w = jnp.concatenate([pro_edge_weight, jnp.ones((N,), dtype=pro_edge_weight.dtype)])
    h = _gcn(pro_x, ps, pd, w, params["pro0"], N)
    h = _gat(h, ps, pd, params["pro1"], N, "relu")
    h = _gat(h, ps, pd, params["pro2"], N, "none")
    xt = _gmp(h, pro_batch, B)

    cs, cd = _loops(clique_edge_index, N)
    h = _gat(clique_x, cs, cd, params["clq0"], N, "relu")
    h = _gat(h, cs, cd, params["clq1"], N, "relu")
    h = _gat(h, cs, cd, params["clq2"], N, "none")
    xq = _gmp(h, clique_batch, B)

    return _head(x, xt, xq, params)


# speedup vs baseline: 30.4866x; 30.4866x over previous
import dataclasses
import functools

import jax
import jax.numpy as jnp
from jax import lax
from jax.experimental import pallas as pl
from jax.experimental.pallas import tpu as pltpu
from jax.experimental.pallas import tpu_sc as plsc

N = 10000
E = 160000
B = 64
H = 2

EP = E + N
NSUB = 16
KB = 128
NB = 84
EPAD = NSUB * NB * KB
NP = 10240
NPS = NP // NSUB

def _mesh():
    return plsc.VectorSubcoreMesh(core_axis_name="c", subcore_axis_name="s",
                                  num_cores=2, num_subcores=NSUB)


def _sc_params():
    cp = pltpu.CompilerParams()
    if "needs_layout_passes" in pltpu.CompilerParams.__dataclass_fields__:
        cp = dataclasses.replace(cp, needs_layout_passes=False)
    return cp


def _spmm_body(mode, chunks, h_hbm, tab_hbm, src_hbm, dst_hbm, val_hbm,
               u_hbm, att_hbm, slab, sem_a, sem_b):
    core = lax.axis_index("c")
    s = lax.axis_index("s")
    NBG = NB // 8

    def phase1(tab_v, sbuf, dbuf, vbuf, a0buf, a1buf):
        pltpu.sync_copy(tab_hbm, tab_v)
        if mode == "gat":
            acc0 = jnp.full((16,), -3.4e38, jnp.float32)
            acc = lax.fori_loop(
                0, (4 * N) // 16,
                lambda i, a: jnp.maximum(a, tab_v[pl.ds(i * 16, 16)]), acc0)
            ms0 = jnp.maximum(jnp.maximum(acc[0], acc[4]),
                              jnp.maximum(acc[8], acc[12]))
            ms1 = jnp.maximum(jnp.maximum(acc[1], acc[5]),
                              jnp.maximum(acc[9], acc[13]))
            mt0 = jnp.maximum(jnp.maximum(acc[2], acc[6]),
                              jnp.maximum(acc[10], acc[14]))
            mt1 = jnp.maximum(jnp.maximum(acc[3], acc[7]),
                              jnp.maximum(acc[11], acc[15]))
            c0 = ms0 + mt0
            c0 = jnp.maximum(c0, 0.2 * c0)
            c1 = ms1 + mt1
            c1 = jnp.maximum(c1, 0.2 * c1)

        @pl.loop(0, NBG)
        def _grp(grp):
            gsl = pl.ds(grp * 8, 8)
            pltpu.sync_copy(src_hbm.at[s, gsl], sbuf)
            pltpu.sync_copy(dst_hbm.at[s, gsl], dbuf)
            pltpu.sync_copy(val_hbm.at[s, gsl], vbuf)
            for bb in range(8):
                @pl.loop(0, KB // 16)
                def _gloop(g):
                    sl = pl.ds(g * 16, 16)
                    if mode == "gat":
                        sv = sbuf[bb, sl] * 4
                        dv = dbuf[bb, sl] * 4
                        vv = vbuf[bb, sl]
                        s0 = plsc.load_gather(tab_v, [sv])
                        s1 = plsc.load_gather(tab_v, [sv + 1])
                        t0 = plsc.load_gather(tab_v, [dv + 2])
                        t1 = plsc.load_gather(tab_v, [dv + 3])
                        a0 = s0 + t0
                        a0 = jnp.maximum(a0, 0.2 * a0)
                        a1 = s1 + t1
                        a1 = jnp.maximum(a1, 0.2 * a1)
                        a0buf[bb, sl] = jnp.exp(a0 - c0) * vv
                        a1buf[bb, sl] = jnp.exp(a1 - c1) * vv
                    else:
                        sv = sbuf[bb, sl]
                        dv = dbuf[bb, sl]
                        wv = vbuf[bb, sl]
                        di_s = plsc.load_gather(tab_v, [sv])
                        di_d = plsc.load_gather(tab_v, [dv])
                        a0buf[bb, sl] = di_s * di_d * wv
            pltpu.sync_copy(a0buf, att_hbm.at[s, 0, gsl])
            if mode == "gat":
                pltpu.sync_copy(a1buf, att_hbm.at[s, 1, gsl])

    tabn = 4 * NP if mode == "gat" else NP
    pl.run_scoped(phase1,
                  pltpu.VMEM((tabn,), jnp.float32),
                  pltpu.VMEM((8, KB), jnp.int32),
                  pltpu.VMEM((8, KB), jnp.int32),
                  pltpu.VMEM((8, KB), jnp.float32),
                  pltpu.VMEM((8, KB), jnp.float32),
                  pltpu.VMEM((8, KB), jnp.float32))

    def phase2(gbuf_a, gbuf_b, sbuf, dbuf, abuf):
        gbufs = (gbuf_a, gbuf_b)
        sems = (sem_a, sem_b)
        C = len(chunks)
        npc = C // 2

        @pl.loop(0, C // 2)
        def _chunkloop(i):
            p = 2 * i + core
            if mode == "gat":
                hd = jnp.where(p >= npc, 1, 0).astype(jnp.int32)
            else:
                hd = 0
            @pl.loop(0, KB)
            def _z(j):
                for r in range(8):
                    gbuf_a[j, pl.ds(r * 16, 16)] = jnp.zeros(
                        (16,), jnp.float32)
            for k in range(NPS // KB):
                pltpu.sync_copy(
                    gbuf_a, slab.at[pl.ds(s * NPS + k * KB, KB)])
            plsc.subcore_barrier()

            def start(bb, buf, sem):
                pltpu.async_copy(
                    h_hbm.at[p].at[sbuf.at[bb]], buf, sem)

            def wait(buf, sem):
                pltpu.make_async_copy(
                    h_hbm.at[p].at[pl.ds(0, KB)], buf, sem).wait()

            def process(bb, buf):
                @pl.loop(0, KB // 16)
                def _gloop(g):
                    av = abuf[bb, pl.ds(g * 16, 16)]
                    for l in range(16):
                        j = g * 16 + l
                        a = av[l]
                        for r in range(8):
                            sl = pl.ds(r * 16, 16)
                            buf[j, sl] = buf[j, sl] * a
                pltpu.sync_copy(buf, slab.at[dbuf.at[bb]], add=True)

            @pl.loop(0, NB // 8)
            def _grp(grp):
                gsl = pl.ds(grp * 8, 8)
                pltpu.sync_copy(src_hbm.at[s, gsl], sbuf)
                pltpu.sync_copy(dst_hbm.at[s, gsl], dbuf)
                pltpu.sync_copy(att_hbm.at[s, hd, gsl], abuf)
                start(0, gbufs[0], sems[0])
                for bb in range(8):
                    if bb < 7:
                        start(bb + 1, gbufs[(bb + 1) % 2],
                              sems[(bb + 1) % 2])
                    wait(gbufs[bb % 2], sems[bb % 2])
                    process(bb, gbufs[bb % 2])

            plsc.subcore_barrier()
            pltpu.sync_copy(slab.at[pl.ds(s * NPS, NPS)],
                            u_hbm.at[p].at[pl.ds(s * NPS, NPS)])
            plsc.subcore_barrier()

    pl.run_scoped(phase2,
                  pltpu.VMEM((KB, 128), jnp.float32),
                  pltpu.VMEM((KB, 128), jnp.float32),
                  pltpu.VMEM((8, KB), jnp.int32),
                  pltpu.VMEM((8, KB), jnp.int32),
                  pltpu.VMEM((8, KB), jnp.float32))


def sc_spmm(h3d, tab, src3, dst3, val3, chunks, mode):
    Cout = h3d.shape[0]
    body = functools.partial(_spmm_body, mode, tuple(chunks))
    f = pl.kernel(
        body,
        out_type=(jax.ShapeDtypeStruct((Cout, NP, 128), jnp.float32),
                  jax.ShapeDtypeStruct((NSUB, 2, NB, KB), jnp.float32)),
        mesh=_mesh(),
        compiler_params=_sc_params(),
        scratch_types=[
            pltpu.VMEM_SHARED((NP, 128), jnp.float32),
            pltpu.SemaphoreType.DMA,
            pltpu.SemaphoreType.DMA,
        ],
    )
    u, _att = f(h3d, tab, src3, dst3, val3)
    return u


def _deg_body(dst_hbm, w_hbm, out_hbm, dst_v, w_v, vbuf, zbuf, slab, sem):
    core = lax.axis_index("c")
    s = lax.axis_index("s")
    nbh = NB // 2
    wid = s * 2 + core
    pltpu.sync_copy(dst_hbm.at[wid], dst_v)
    pltpu.sync_copy(w_hbm.at[wid], w_v)

    iot = lax.iota(jnp.int32, 16)
    zc = jnp.zeros((16,), jnp.int32)

    @pl.loop(0, KB)
    def _z(j):
        for r in range(8):
            zbuf[j, pl.ds(r * 16, 16)] = jnp.zeros((16,), jnp.float32)
            vbuf[j, pl.ds(r * 16, 16)] = jnp.zeros((16,), jnp.float32)

    for k in range(NPS // 128):
        pltpu.sync_copy(zbuf, slab.at[pl.ds(s * NPS + k * 128, 128)])
    plsc.subcore_barrier()

    @pl.loop(0, nbh)
    def _bloop(b):
        @pl.loop(0, KB // 16)
        def _g(g):
            wv = w_v[b, pl.ds(g * 16, 16)]
            plsc.store_scatter(vbuf, [g * 16 + iot, zc], wv)
        pltpu.sync_copy(vbuf, slab.at[dst_v.at[b]], add=True)

    plsc.subcore_barrier()
    pltpu.sync_copy(slab.at[pl.ds(s * NPS, NPS)],
                    out_hbm.at[core].at[pl.ds(s * NPS, NPS)])


def sc_deg(dst3, w3):
    dst3 = dst3.reshape(2 * NSUB, NB // 2, KB)
    w3 = w3.reshape(2 * NSUB, NB // 2, KB)
    f = pl.kernel(
        _deg_body,
        out_type=jax.ShapeDtypeStruct((2, NP, 128), jnp.float32),
        mesh=_mesh(),
        compiler_params=_sc_params(),
        scratch_types=[
            pltpu.VMEM((NB // 2, KB), jnp.int32),
            pltpu.VMEM((NB // 2, KB), jnp.float32),
            pltpu.VMEM((KB, 128), jnp.float32),
            pltpu.VMEM((KB, 128), jnp.float32),
            pltpu.VMEM_SHARED((NP, 128), jnp.float32),
            pltpu.SemaphoreType.DMA,
        ],
    )
    return f(dst3, w3)


def _mm_body(npairs, Cout, mode, dpos, x_ref, w_ref, b_ref, ones_ref, wa_ref,
             o_ref, st_ref):
    if mode == "in2d":
        x = x_ref[...]
        xs = [x]
    else:
        if mode == "in3d_gat":
            (c0, l0), (c1, l1) = dpos
            den0 = x_ref[c0][:, l0:l0 + 1]
            den1 = x_ref[c1][:, l1:l1 + 1]
            rd0 = 0.5 / (den0 + 1e-16)
            rd1 = 0.5 / (den1 + 1e-16)
        xs = []
        for i in range(npairs):
            bi = b_ref[:, i * 128:(i + 1) * 128]
            if mode == "in3d_gat":
                xi = x_ref[i] * rd0 + x_ref[npairs + i] * rd1 + bi
            else:
                xi = x_ref[i] + bi
            xs.append(jnp.maximum(xi, 0.0))
    acc = jnp.zeros((x_ref.shape[-2], Cout * 128), jnp.float32)
    for i, xi in enumerate(xs):
        if mode == "in2d":
            wi = w_ref[...]
        else:
            wi = w_ref[pl.ds(i * 128, 128), :]
        acc = acc + jnp.dot(xi, wi, preferred_element_type=jnp.float32)
    acc = acc + ones_ref[...]
    for c in range(Cout):
        o_ref[c] = acc[:, c * 128:(c + 1) * 128]
    if st_ref is not None:
        sacc = jnp.zeros((x_ref.shape[-2], wa_ref.shape[-1]), jnp.float32)
        for i, xi in enumerate(xs):
            if mode == "in2d":
                wai = wa_ref[...]
            else:
                wai = wa_ref[pl.ds(i * 128, 128), :]
            sacc = sacc + jnp.dot(xi, wai, preferred_element_type=jnp.float32)
        st_ref[...] = sacc


@functools.partial(jax.jit, static_argnames=("mode", "dpos", "want_st", "block_m"))
def tc_project(x, w, bias, ones, wa, mode, dpos=None, want_st=True, block_m=640):
    if mode == "in2d":
        npairs = 0
        xspec = pl.BlockSpec((block_m, x.shape[1]), lambda i: (i, 0))
    else:
        npairs = w.shape[0] // 128
        xspec = pl.BlockSpec((x.shape[0], block_m, 128), lambda i: (0, i, 0))
    Cout = w.shape[1] // 128
    nb = NP // block_m
    out_shapes = [jax.ShapeDtypeStruct((Cout, NP, 128), jnp.float32)]
    out_specs = [pl.BlockSpec((Cout, block_m, 128), lambda i: (0, i, 0))]
    if want_st:
        out_shapes.append(jax.ShapeDtypeStruct((NP, 4), jnp.float32))
        out_specs.append(pl.BlockSpec((block_m, 4), lambda i: (i, 0)))
    body = functools.partial(_mm_body, npairs, Cout, mode, dpos)
    if not want_st:
        body = functools.partial(
            lambda f, *refs: f(*refs, None), body)
    res = pl.pallas_call(
        body,
        grid=(nb,),
        in_specs=[
            xspec,
            pl.BlockSpec(w.shape, lambda i: tuple(0 for _ in w.shape)),
            pl.BlockSpec(bias.shape, lambda i: (0, 0)),
            pl.BlockSpec(ones.shape, lambda i: (0, 0)),
            pl.BlockSpec(wa.shape, lambda i: (0, 0)),
        ],
        out_specs=out_specs,
        out_shape=out_shapes,
    )(x, w, bias, ones, wa)
    return res if want_st else res[0]


def _gmp_body(pairs, dpos, NW, NPW, h_hbm, batch_hbm, bias_hbm, out_hbm,
              bid_v, bias_v, rd0_v, rd1_v, acc_v, stage, sem):
    core = lax.axis_index("c")
    s = lax.axis_index("s")
    wid = s * 2 + core
    npair = len(pairs)
    W = npair * 128
    pltpu.sync_copy(bias_hbm, bias_v)

    ninf = jnp.full((16,), -jnp.inf, jnp.float32)

    @pl.loop(0, B)
    def _ai(j):
        for r in range(W // 16):
            acc_v[j, pl.ds(r * 16, 16)] = ninf

    NH = NPW // 2

    def work(dsl_v, dsl2_v):
        (c0, l0), (c1, l1) = dpos
        iot = lax.iota(jnp.int32, 16)
        pltpu.sync_copy(batch_hbm.at[pl.ds(wid * NPW, NPW)], bid_v)
        for half in range(2):
            base = wid * NPW + half * NH
            hoff = half * NH
            pltpu.sync_copy(h_hbm.at[c0].at[pl.ds(base, NH)], dsl_v)
            pltpu.sync_copy(h_hbm.at[c1].at[pl.ds(base, NH)], dsl2_v)

            @pl.loop(0, NH // 16)
            def _rd(g):
                rows = g * 16 + iot
                d0 = plsc.load_gather(
                    dsl_v, [rows, jnp.full((16,), l0, jnp.int32)])
                d1 = plsc.load_gather(
                    dsl2_v, [rows, jnp.full((16,), l1, jnp.int32)])
                rd0_v[pl.ds(hoff + g * 16, 16)] = 0.5 / (d0 + 1e-16)
                rd1_v[pl.ds(hoff + g * 16, 16)] = 0.5 / (d1 + 1e-16)

        @pl.loop(0, npair)
        def _pair(ip):
            @pl.loop(0, 2)
            def _half(half):
                base = wid * NPW + half * NH
                hoff = half * NH
                pltpu.sync_copy(h_hbm.at[ip].at[pl.ds(base, NH)], dsl_v)
                pltpu.sync_copy(h_hbm.at[npair + ip].at[pl.ds(base, NH)],
                                dsl2_v)

                @pl.loop(0, NH // 16)
                def _nl(g):
                    bv = bid_v[pl.ds(hoff + g * 16, 16)]
                    av = rd0_v[pl.ds(hoff + g * 16, 16)]
                    cv = rd1_v[pl.ds(hoff + g * 16, 16)]
                    mf = jnp.where((base + g * 16 + iot) < N, 0.0,
                                   -jnp.inf).astype(jnp.float32)
                    for l in range(16):
                        j = g * 16 + l
                        bid = bv[l]
                        a = av[l]
                        c = cv[l]
                        mfl = mf[l]
                        for r in range(8):
                            sl = pl.ds(r * 16, 16)
                            osl = pl.ds(ip * 128 + r * 16, 16)
                            x = (dsl_v[j, sl] * a + dsl2_v[j, sl] * c
                                 + bias_v[osl] + mfl)
                            acc_v[bid, osl] = jnp.maximum(acc_v[bid, osl], x)

    pl.run_scoped(work, pltpu.VMEM((NPW // 2, 128), jnp.float32),
                  pltpu.VMEM((NPW // 2, 128), jnp.float32))

    pltpu.sync_copy(acc_v, stage.at[s])
    plsc.subcore_barrier()

    def combine(cmb_v):
        @pl.when(s < 8)
        def _comb_band():
            for t in range(NSUB):
                pltpu.sync_copy(stage.at[t, pl.ds(s * 8, 8)], cmb_v.at[t])

            @pl.loop(0, 8)
            def _comb(r):
                for r2 in range(W // 16):
                    sl = pl.ds(r2 * 16, 16)
                    m = cmb_v[0, r, sl]
                    for t in range(1, NSUB):
                        m = jnp.maximum(m, cmb_v[t, r, sl])
                    acc_v[r, sl] = m

            pltpu.sync_copy(acc_v.at[pl.ds(0, 8)],
                            out_hbm.at[core].at[pl.ds(s * 8, 8)])

    pl.run_scoped(combine, pltpu.VMEM((NSUB, 8, W), jnp.float32))


def sc_gmp(h3d, batch, bias, pairs, dpos):
    NW = 32
    NPW = NP // NW
    W = len(pairs) * 128
    body = functools.partial(_gmp_body, tuple(pairs), tuple(dpos), NW, NPW)
    f = pl.kernel(
        body,
        out_type=jax.ShapeDtypeStruct((2, B, W), jnp.float32),
        mesh=_mesh(),
        compiler_params=_sc_params(),
        scratch_types=[
            pltpu.VMEM((NPW,), jnp.int32),
            pltpu.VMEM((W,), jnp.float32),
            pltpu.VMEM((NPW,), jnp.float32),
            pltpu.VMEM((NPW,), jnp.float32),
            pltpu.VMEM((B, W), jnp.float32),
            pltpu.VMEM_SHARED((NSUB, B, W), jnp.float32),
            pltpu.SemaphoreType.DMA,
        ],
    )
    return f(h3d, batch, bias)


def _dinv_body(p_ref, o_ref):
    d = p_ref[0] + p_ref[1]
    o_ref[...] = jnp.where(d > 0, lax.rsqrt(d), 0.0)


def tc_dinv(partials):
    return pl.pallas_call(
        _dinv_body,
        out_shape=jax.ShapeDtypeStruct((NP,), jnp.float32),
    )(partials)


def _head_body(fouts, x_ref, xt_ref, xq_ref, *refs):
    (mf1w, mf1b, mf2w, mf2b, pf1w, pf1b, pf2w, pf2b, cf1w, cf1b, cf2w, cf2b,
     ax1, axb, ax2, at1, atb, at2, aq1, aqb, aq2,
     f1w, f1b, f2w, f2b, ow, ob, o_ref) = refs
    dot = lambda a, b: jnp.dot(a, b, preferred_element_type=jnp.float32)
    relu = lambda v: jnp.maximum(v, 0.0)

    def pool(ref, fout):
        m = jnp.maximum(ref[0], ref[1])[:, :fout]
        return jnp.where(jnp.isfinite(m), m, 0.0)

    x = pool(x_ref, fouts[0])
    xt = pool(xt_ref, fouts[1])
    xq = pool(xq_ref, fouts[2])
    x = dot(relu(dot(x, mf1w[...]) + mf1b[...]), mf2w[...]) + mf2b[...]
    xt = dot(relu(dot(xt, pf1w[...]) + pf1b[...]), pf2w[...]) + pf2b[...]
    xq = dot(relu(dot(xq, cf1w[...]) + cf1b[...]), cf2w[...]) + cf2b[...]
    ax = dot(jnp.tanh(dot(x, ax1[...]) + axb[...]), ax2[...])
    at = dot(jnp.tanh(dot(xt, at1[...]) + atb[...]), at2[...])
    aq = dot(jnp.tanh(dot(xq, aq1[...]) + aqb[...]), aq2[...])
    a = jnp.concatenate([ax, at, aq], axis=1)
    a = jax.nn.softmax(a, axis=1)
    emb = jnp.concatenate(
        [a[:, 0:1] * x, a[:, 1:2] * xt, a[:, 2:3] * xq], axis=1)
    xc = relu(dot(emb, f1w[...]) + f1b[...])
    xc = relu(dot(xc, f2w[...]) + f2b[...])
    o_ref[...] = dot(xc, ow[...]) + ob[...]


def tc_head(x, xt, xq, params, fouts):
    args = [x, xt, xq]
    for nm in ("mol_fc1", "mol_fc2", "pro_fc1", "pro_fc2", "clq_fc1", "clq_fc2"):
        args += [params[nm]["W"], params[nm]["b"].reshape(1, -1)]
    for nm in ("att_x", "att_xt", "att_xq"):
        args += [params[nm]["W1"], params[nm]["b1"].reshape(1, -1), params[nm]["W2"]]
    for nm in ("fc1", "fc2", "out"):
        args += [params[nm]["W"], params[nm]["b"].reshape(1, -1)]
    return pl.pallas_call(
        functools.partial(_head_body, tuple(fouts)),
        out_shape=jax.ShapeDtypeStruct((B, 1), jnp.float32),
    )(*args)


def _gat_layout(fout):
    hp = ((fout + 1 + 127) // 128) * 128
    npc = hp // 128
    C = 2 * npc
    chunks = [(i, 0) for i in range(npc)] + [(npc + i, 1) for i in range(npc)]
    dch = fout // 128
    dlane = fout - dch * 128
    return dict(fout=fout, hp=hp, npc=npc, C=C, chunks=chunks,
                dpos=((dch, dlane), (npc + dch, dlane)),
                pairs=[(i, npc + i) for i in range(npc)])


def _pack_gat(prm, lay):
    fout, hp, npc = lay["fout"], lay["hp"], lay["npc"]
    Wf = prm["W"]
    fin = Wf.shape[0]
    W0, W1 = Wf[:, :fout], Wf[:, fout:]
    z = jnp.zeros((fin, hp - fout), jnp.float32)
    w = jnp.concatenate([W0, z, W1, z], axis=1)
    ones = jnp.zeros((2 * hp,), jnp.float32)
    ones = ones.at[fout].set(1.0).at[hp + fout].set(1.0).reshape(1, -1)
    wa = jnp.concatenate([
        (W0 @ prm["asrc"][0])[:, None], (W1 @ prm["asrc"][1])[:, None],
        (W0 @ prm["adst"][0])[:, None], (W1 @ prm["adst"][1])[:, None]],
        axis=1)
    bias = jnp.concatenate([prm["b"], jnp.zeros((hp - fout,), jnp.float32)])
    return w, ones, wa, bias


def _pad_rows(w, rows):
    fin = w.shape[0]
    if fin == rows:
        return w
    return jnp.concatenate(
        [w, jnp.zeros((rows - fin,) + w.shape[1:], jnp.float32)], axis=0)


def _edges3(edge_index, extra_w=None):
    src = jnp.concatenate([edge_index[0], jnp.arange(N, dtype=jnp.int32),
                           jnp.zeros((EPAD - EP,), jnp.int32)]).astype(jnp.int32)
    dst = jnp.concatenate([edge_index[1], jnp.arange(N, dtype=jnp.int32),
                           jnp.zeros((EPAD - EP,), jnp.int32)]).astype(jnp.int32)
    val = jnp.concatenate([jnp.ones((EP,), jnp.float32),
                           jnp.zeros((EPAD - EP,), jnp.float32)])
    if extra_w is not None:
        val = jnp.concatenate([extra_w, jnp.ones((N,), jnp.float32),
                               jnp.zeros((EPAD - EP,), jnp.float32)])
    r = lambda a: a.reshape(NSUB, NB, KB)
    return r(src), r(dst), r(val)


def _gat_layer(x_or_u, prm, lay_in, lay, src3, dst3, val3, mode):
    w, ones, wa, bias = _pack_gat(prm, lay)
    if mode == "in2d":
        wk = w
        wak = wa
        bk = jnp.zeros((1, 64), jnp.float32)
    else:
        rows = lay_in["npc"] * 128 if lay_in is not None else None
        wk = _pad_rows(w, rows)
        wak = _pad_rows(wa, rows)
        bk = lay_in["bias"].reshape(1, -1)
    h3d, st = tc_project(x_or_u, wk, bk, ones, wak, mode,
                         dpos=lay_in["dpos"] if mode == "in3d_gat" else None,
                         want_st=True)
    u = sc_spmm(h3d, st.reshape(-1), src3, dst3, val3, lay["chunks"], "gat")
    return u


def kernel(mol_x, pro_x, clique_x, pro_edge_weight, params, mol_edge_index,
           pro_edge_index, clique_edge_index, mol_batch, pro_batch,
           clique_batch):
    mol_edge_index = mol_edge_index.astype(jnp.int32)
    pro_edge_index = pro_edge_index.astype(jnp.int32)
    clique_edge_index = clique_edge_index.astype(jnp.int32)
    padb = jnp.zeros((NP - N,), jnp.int32)
    mol_batch = jnp.concatenate([mol_batch.astype(jnp.int32), padb])
    pro_batch = jnp.concatenate([pro_batch.astype(jnp.int32), padb])
    clique_batch = jnp.concatenate([clique_batch.astype(jnp.int32), padb])

    lay_m = _gat_layout(312)
    lay_p = _gat_layout(132)
    lay_q = _gat_layout(368)

    s3, d3, v3 = _edges3(mol_edge_index)
    u = _gat_layer(mol_x, params["mol0"], None, lay_m, s3, d3, v3, "in2d")
    u = _gat_layer(u, params["mol1"],
                   dict(lay_m, bias=_pack_gat(params["mol0"], lay_m)[3]),
                   lay_m, s3, d3, v3, "in3d_gat")
    u = _gat_layer(u, params["mol2"],
                   dict(lay_m, bias=_pack_gat(params["mol1"], lay_m)[3]),
                   lay_m, s3, d3, v3, "in3d_gat")
    bias_last = jnp.concatenate(
        [params["mol2"]["b"],
         jnp.zeros((lay_m["hp"] - 312,), jnp.float32)])
    xp = sc_gmp(u, mol_batch, bias_last, lay_m["pairs"], lay_m["dpos"])

    s3p, d3p, v3p = _edges3(pro_edge_index)
    _, _, w3p = _edges3(pro_edge_index, extra_w=pro_edge_weight)
    partials = sc_deg(d3p, w3p)
    dinv = tc_dinv(partials[:, :, 0])
    hp_g = ((132 + 127) // 128) * 128
    Wg = jnp.concatenate(
        [params["pro0"]["W"],
         jnp.zeros((33, hp_g - 132), jnp.float32)], axis=1)
    zeros_ones = jnp.zeros((1, hp_g), jnp.float32)
    dummy_wa = jnp.zeros((33, 4), jnp.float32)
    hg = tc_project(pro_x, Wg, jnp.zeros((1, 64), jnp.float32), zeros_ones,
                    dummy_wa, "in2d", want_st=False)
    gchunks = [(i, 0) for i in range(hp_g // 128)]
    ug = sc_spmm(hg, dinv, s3p, d3p, w3p, gchunks, "gcn")
    bias_g = jnp.concatenate(
        [params["pro0"]["b"], jnp.zeros((hp_g - 132,), jnp.float32)])
    w1, ones1, wa1, bias1 = _pack_gat(params["pro1"], lay_p)
    h3d, st = tc_project(ug, _pad_rows(w1, hp_g), bias_g.reshape(1, -1),
                         ones1, _pad_rows(wa1, hp_g), "in3d_gcn",
                         want_st=True)
    u = sc_spmm(h3d, st.reshape(-1), s3p, d3p, v3p, lay_p["chunks"], "gat")
    u = _gat_layer(u, params["pro2"], dict(lay_p, bias=bias1), lay_p,
                   s3p, d3p, v3p, "in3d_gat")
    bias_last = jnp.concatenate(
        [params["pro2"]["b"], jnp.zeros((lay_p["hp"] - 132,), jnp.float32)])
    xtp = sc_gmp(u, pro_batch, bias_last, lay_p["pairs"], lay_p["dpos"])

    s3c, d3c, v3c = _edges3(clique_edge_index)
    u = _gat_layer(clique_x, params["clq0"], None, lay_q, s3c, d3c, v3c, "in2d")
    u = _gat_layer(u, params["clq1"],
                   dict(lay_q, bias=_pack_gat(params["clq0"], lay_q)[3]),
                   lay_q, s3c, d3c, v3c, "in3d_gat")
    u = _gat_layer(u, params["clq2"],
                   dict(lay_q, bias=_pack_gat(params["clq1"], lay_q)[3]),
                   lay_q, s3c, d3c, v3c, "in3d_gat")
    bias_last = jnp.concatenate(
        [params["clq2"]["b"], jnp.zeros((lay_q["hp"] - 368,), jnp.float32)])
    xqp = sc_gmp(u, clique_batch, bias_last, lay_q["pairs"], lay_q["dpos"])

    return tc_head(xp, xtp, xqp, params, (312, 132, 368))


# speedup vs baseline: 32.6741x; 1.0718x over previous
import dataclasses
import functools

import jax
import jax.numpy as jnp
from jax import lax
from jax.experimental import pallas as pl
from jax.experimental.pallas import tpu as pltpu
from jax.experimental.pallas import tpu_sc as plsc

N = 10000
E = 160000
B = 64
H = 2

EP = E + N
NSUB = 16
KB = 128
NB = 84
EPAD = NSUB * NB * KB
NP = 10240
NPS = NP // NSUB

def _mesh():
    return plsc.VectorSubcoreMesh(core_axis_name="c", subcore_axis_name="s",
                                  num_cores=2, num_subcores=NSUB)


def _sc_params():
    cp = pltpu.CompilerParams()
    if "needs_layout_passes" in pltpu.CompilerParams.__dataclass_fields__:
        cp = dataclasses.replace(cp, needs_layout_passes=False)
    return cp


def _spmm_body(mode, chunks, h_hbm, tab_hbm, src_hbm, dst_hbm, val_hbm,
               u_hbm, att_hbm, slab, sem_a, sem_b, sem_c, sem_d, sem_e):
    core = lax.axis_index("c")
    s = lax.axis_index("s")
    NBG = NB // 8

    def phase1(tab_v, sbuf, dbuf, vbuf, a0buf, a1buf):
        pltpu.sync_copy(tab_hbm, tab_v)
        if mode == "gat":
            acc0 = jnp.full((16,), -3.4e38, jnp.float32)
            acc = lax.fori_loop(
                0, (4 * N) // 16,
                lambda i, a: jnp.maximum(a, tab_v[pl.ds(i * 16, 16)]), acc0)
            ms0 = jnp.maximum(jnp.maximum(acc[0], acc[4]),
                              jnp.maximum(acc[8], acc[12]))
            ms1 = jnp.maximum(jnp.maximum(acc[1], acc[5]),
                              jnp.maximum(acc[9], acc[13]))
            mt0 = jnp.maximum(jnp.maximum(acc[2], acc[6]),
                              jnp.maximum(acc[10], acc[14]))
            mt1 = jnp.maximum(jnp.maximum(acc[3], acc[7]),
                              jnp.maximum(acc[11], acc[15]))
            c0 = ms0 + mt0
            c0 = jnp.maximum(c0, 0.2 * c0)
            c1 = ms1 + mt1
            c1 = jnp.maximum(c1, 0.2 * c1)

        def wait_att_stores():
            pltpu.make_async_copy(a0buf, att_hbm.at[s, 0, pl.ds(0, 8)],
                                  sem_b).wait()
            if mode == "gat":
                pltpu.make_async_copy(a1buf, att_hbm.at[s, 1, pl.ds(0, 8)],
                                      sem_b).wait()

        @pl.loop(0, NBG)
        def _grp(grp):
            gsl = pl.ds(grp * 8, 8)
            d1 = pltpu.async_copy(src_hbm.at[s, gsl], sbuf, sem_a)
            d2 = pltpu.async_copy(dst_hbm.at[s, gsl], dbuf, sem_a)
            d3 = pltpu.async_copy(val_hbm.at[s, gsl], vbuf, sem_a)
            d1.wait()
            d2.wait()
            d3.wait()

            @pl.when(grp > 0)
            def _():
                wait_att_stores()
            for bb in range(8):
                @pl.loop(0, KB // 16)
                def _gloop(g):
                    sl = pl.ds(g * 16, 16)
                    if mode == "gat":
                        sv = sbuf[bb, sl] * 4
                        dv = dbuf[bb, sl] * 4
                        vv = vbuf[bb, sl]
                        s0 = plsc.load_gather(tab_v, [sv])
                        s1 = plsc.load_gather(tab_v, [sv + 1])
                        t0 = plsc.load_gather(tab_v, [dv + 2])
                        t1 = plsc.load_gather(tab_v, [dv + 3])
                        a0 = s0 + t0
                        a0 = jnp.maximum(a0, 0.2 * a0)
                        a1 = s1 + t1
                        a1 = jnp.maximum(a1, 0.2 * a1)
                        a0buf[bb, sl] = jnp.exp(a0 - c0) * vv
                        a1buf[bb, sl] = jnp.exp(a1 - c1) * vv
                    else:
                        sv = sbuf[bb, sl]
                        dv = dbuf[bb, sl]
                        wv = vbuf[bb, sl]
                        di_s = plsc.load_gather(tab_v, [sv])
                        di_d = plsc.load_gather(tab_v, [dv])
                        a0buf[bb, sl] = di_s * di_d * wv
            pltpu.async_copy(a0buf, att_hbm.at[s, 0, gsl], sem_b)
            if mode == "gat":
                pltpu.async_copy(a1buf, att_hbm.at[s, 1, gsl], sem_b)

        wait_att_stores()

    tabn = 4 * NP if mode == "gat" else NP
    pl.run_scoped(phase1,
                  pltpu.VMEM((tabn,), jnp.float32),
                  pltpu.VMEM((8, KB), jnp.int32),
                  pltpu.VMEM((8, KB), jnp.int32),
                  pltpu.VMEM((8, KB), jnp.float32),
                  pltpu.VMEM((8, KB), jnp.float32),
                  pltpu.VMEM((8, KB), jnp.float32))

    def phase2(gbuf_a, gbuf_b, sbuf, dbuf, abuf):
        gbufs = (gbuf_a, gbuf_b)
        sems = (sem_a, sem_b)
        C = len(chunks)
        npc = C // 2

        @pl.loop(0, C // 2)
        def _chunkloop(i):
            p = 2 * i + core
            if mode == "gat":
                hd = jnp.where(p >= npc, 1, 0).astype(jnp.int32)
            else:
                hd = 0
            @pl.loop(0, KB)
            def _z(j):
                for r in range(8):
                    gbuf_a[j, pl.ds(r * 16, 16)] = jnp.zeros(
                        (16,), jnp.float32)
            for k in range(NPS // KB):
                pltpu.sync_copy(
                    gbuf_a, slab.at[pl.ds(s * NPS + k * KB, KB)])
            plsc.subcore_barrier()

            def start(bb, buf, sem):
                pltpu.async_copy(
                    h_hbm.at[p].at[sbuf.at[bb]], buf, sem)

            def wait(buf, sem):
                pltpu.make_async_copy(
                    h_hbm.at[p].at[pl.ds(0, KB)], buf, sem).wait()

            scsems = (sem_c, sem_d)

            def wait_scatter(x):
                pltpu.make_async_copy(h_hbm.at[p].at[pl.ds(0, KB)],
                                      gbufs[x], scsems[x]).wait()

            def process(bb, buf, scs):
                @pl.loop(0, KB // 16)
                def _gloop(g):
                    av = abuf[bb, pl.ds(g * 16, 16)]
                    for l in range(16):
                        j = g * 16 + l
                        a = av[l]
                        for r in range(8):
                            sl = pl.ds(r * 16, 16)
                            buf[j, sl] = buf[j, sl] * a
                pltpu.async_copy(buf, slab.at[dbuf.at[bb]], scs, add=True)

            @pl.loop(0, NB // 8)
            def _grp(grp):
                gsl = pl.ds(grp * 8, 8)
                d1 = pltpu.async_copy(src_hbm.at[s, gsl], sbuf, sem_e)
                d2 = pltpu.async_copy(dst_hbm.at[s, gsl], dbuf, sem_e)
                d3 = pltpu.async_copy(att_hbm.at[s, hd, gsl], abuf, sem_e)
                d1.wait()
                d2.wait()
                d3.wait()

                @pl.when(grp > 0)
                def _():
                    wait_scatter(0)
                start(0, gbufs[0], sems[0])
                for bb in range(8):
                    if bb < 7:
                        if bb == 0:
                            @pl.when(grp > 0)
                            def _():
                                wait_scatter(1)
                        else:
                            wait_scatter((bb + 1) % 2)
                        start(bb + 1, gbufs[(bb + 1) % 2],
                              sems[(bb + 1) % 2])
                    wait(gbufs[bb % 2], sems[bb % 2])
                    process(bb, gbufs[bb % 2], scsems[bb % 2])

            wait_scatter(0)
            wait_scatter(1)
            plsc.subcore_barrier()
            pltpu.sync_copy(slab.at[pl.ds(s * NPS, NPS)],
                            u_hbm.at[p].at[pl.ds(s * NPS, NPS)])
            plsc.subcore_barrier()

    pl.run_scoped(phase2,
                  pltpu.VMEM((KB, 128), jnp.float32),
                  pltpu.VMEM((KB, 128), jnp.float32),
                  pltpu.VMEM((8, KB), jnp.int32),
                  pltpu.VMEM((8, KB), jnp.int32),
                  pltpu.VMEM((8, KB), jnp.float32))


def sc_spmm(h3d, tab, src3, dst3, val3, chunks, mode):
    Cout = h3d.shape[0]
    body = functools.partial(_spmm_body, mode, tuple(chunks))
    f = pl.kernel(
        body,
        out_type=(jax.ShapeDtypeStruct((Cout, NP, 128), jnp.float32),
                  jax.ShapeDtypeStruct((NSUB, 2, NB, KB), jnp.float32)),
        mesh=_mesh(),
        compiler_params=_sc_params(),
        scratch_types=[
            pltpu.VMEM_SHARED((NP, 128), jnp.float32),
            pltpu.SemaphoreType.DMA,
            pltpu.SemaphoreType.DMA,
            pltpu.SemaphoreType.DMA,
            pltpu.SemaphoreType.DMA,
            pltpu.SemaphoreType.DMA,
        ],
    )
    u, _att = f(h3d, tab, src3, dst3, val3)
    return u


def _deg_body(dst_hbm, w_hbm, out_hbm, dst_v, w_v, vbuf, zbuf, slab, sem):
    core = lax.axis_index("c")
    s = lax.axis_index("s")
    nbh = NB // 2
    wid = s * 2 + core
    pltpu.sync_copy(dst_hbm.at[wid], dst_v)
    pltpu.sync_copy(w_hbm.at[wid], w_v)

    iot = lax.iota(jnp.int32, 16)
    zc = jnp.zeros((16,), jnp.int32)

    @pl.loop(0, KB)
    def _z(j):
        for r in range(8):
            zbuf[j, pl.ds(r * 16, 16)] = jnp.zeros((16,), jnp.float32)
            vbuf[j, pl.ds(r * 16, 16)] = jnp.zeros((16,), jnp.float32)

    for k in range(NPS // 128):
        pltpu.sync_copy(zbuf, slab.at[pl.ds(s * NPS + k * 128, 128)])
    plsc.subcore_barrier()

    @pl.loop(0, nbh)
    def _bloop(b):
        @pl.loop(0, KB // 16)
        def _g(g):
            wv = w_v[b, pl.ds(g * 16, 16)]
            plsc.store_scatter(vbuf, [g * 16 + iot, zc], wv)
        pltpu.sync_copy(vbuf, slab.at[dst_v.at[b]], add=True)

    plsc.subcore_barrier()
    pltpu.sync_copy(slab.at[pl.ds(s * NPS, NPS)],
                    out_hbm.at[core].at[pl.ds(s * NPS, NPS)])


def sc_deg(dst3, w3):
    dst3 = dst3.reshape(2 * NSUB, NB // 2, KB)
    w3 = w3.reshape(2 * NSUB, NB // 2, KB)
    f = pl.kernel(
        _deg_body,
        out_type=jax.ShapeDtypeStruct((2, NP, 128), jnp.float32),
        mesh=_mesh(),
        compiler_params=_sc_params(),
        scratch_types=[
            pltpu.VMEM((NB // 2, KB), jnp.int32),
            pltpu.VMEM((NB // 2, KB), jnp.float32),
            pltpu.VMEM((KB, 128), jnp.float32),
            pltpu.VMEM((KB, 128), jnp.float32),
            pltpu.VMEM_SHARED((NP, 128), jnp.float32),
            pltpu.SemaphoreType.DMA,
        ],
    )
    return f(dst3, w3)


def _mm_body(npairs, Cout, mode, dpos, x_ref, w_ref, b_ref, ones_ref, wa_ref,
             o_ref, st_ref):
    if mode == "in2d":
        x = x_ref[...]
        xs = [x]
    else:
        if mode == "in3d_gat":
            (c0, l0), (c1, l1) = dpos
            den0 = x_ref[c0][:, l0:l0 + 1]
            den1 = x_ref[c1][:, l1:l1 + 1]
            rd0 = 0.5 / (den0 + 1e-16)
            rd1 = 0.5 / (den1 + 1e-16)
        xs = []
        for i in range(npairs):
            bi = b_ref[:, i * 128:(i + 1) * 128]
            if mode == "in3d_gat":
                xi = x_ref[i] * rd0 + x_ref[npairs + i] * rd1 + bi
            else:
                xi = x_ref[i] + bi
            xs.append(jnp.maximum(xi, 0.0))
    acc = jnp.zeros((x_ref.shape[-2], Cout * 128), jnp.float32)
    for i, xi in enumerate(xs):
        if mode == "in2d":
            wi = w_ref[...]
        else:
            wi = w_ref[pl.ds(i * 128, 128), :]
        acc = acc + jnp.dot(xi, wi, preferred_element_type=jnp.float32)
    acc = acc + ones_ref[...]
    for c in range(Cout):
        o_ref[c] = acc[:, c * 128:(c + 1) * 128]
    if st_ref is not None:
        sacc = jnp.zeros((x_ref.shape[-2], wa_ref.shape[-1]), jnp.float32)
        for i, xi in enumerate(xs):
            if mode == "in2d":
                wai = wa_ref[...]
            else:
                wai = wa_ref[pl.ds(i * 128, 128), :]
            sacc = sacc + jnp.dot(xi, wai, preferred_element_type=jnp.float32)
        st_ref[...] = sacc


@functools.partial(jax.jit, static_argnames=("mode", "dpos", "want_st", "block_m"))
def tc_project(x, w, bias, ones, wa, mode, dpos=None, want_st=True, block_m=640):
    if mode == "in2d":
        npairs = 0
        xspec = pl.BlockSpec((block_m, x.shape[1]), lambda i: (i, 0))
    else:
        npairs = w.shape[0] // 128
        xspec = pl.BlockSpec((x.shape[0], block_m, 128), lambda i: (0, i, 0))
    Cout = w.shape[1] // 128
    nb = NP // block_m
    out_shapes = [jax.ShapeDtypeStruct((Cout, NP, 128), jnp.float32)]
    out_specs = [pl.BlockSpec((Cout, block_m, 128), lambda i: (0, i, 0))]
    if want_st:
        out_shapes.append(jax.ShapeDtypeStruct((NP, 4), jnp.float32))
        out_specs.append(pl.BlockSpec((block_m, 4), lambda i: (i, 0)))
    body = functools.partial(_mm_body, npairs, Cout, mode, dpos)
    if not want_st:
        body = functools.partial(
            lambda f, *refs: f(*refs, None), body)
    res = pl.pallas_call(
        body,
        grid=(nb,),
        in_specs=[
            xspec,
            pl.BlockSpec(w.shape, lambda i: tuple(0 for _ in w.shape)),
            pl.BlockSpec(bias.shape, lambda i: (0, 0)),
            pl.BlockSpec(ones.shape, lambda i: (0, 0)),
            pl.BlockSpec(wa.shape, lambda i: (0, 0)),
        ],
        out_specs=out_specs,
        out_shape=out_shapes,
    )(x, w, bias, ones, wa)
    return res if want_st else res[0]


def _gmp_body(pairs, dpos, NW, NPW, h_hbm, batch_hbm, bias_hbm, out_hbm,
              bid_v, bias_v, rd0_v, rd1_v, acc_v, stage, sem):
    core = lax.axis_index("c")
    s = lax.axis_index("s")
    wid = s * 2 + core
    npair = len(pairs)
    W = npair * 128
    pltpu.sync_copy(bias_hbm, bias_v)

    ninf = jnp.full((16,), -jnp.inf, jnp.float32)

    @pl.loop(0, B)
    def _ai(j):
        for r in range(W // 16):
            acc_v[j, pl.ds(r * 16, 16)] = ninf

    NH = NPW // 2

    def work(dsl_v, dsl2_v):
        (c0, l0), (c1, l1) = dpos
        iot = lax.iota(jnp.int32, 16)
        pltpu.sync_copy(batch_hbm.at[pl.ds(wid * NPW, NPW)], bid_v)
        for half in range(2):
            base = wid * NPW + half * NH
            hoff = half * NH
            pltpu.sync_copy(h_hbm.at[c0].at[pl.ds(base, NH)], dsl_v)
            pltpu.sync_copy(h_hbm.at[c1].at[pl.ds(base, NH)], dsl2_v)

            @pl.loop(0, NH // 16)
            def _rd(g):
                rows = g * 16 + iot
                d0 = plsc.load_gather(
                    dsl_v, [rows, jnp.full((16,), l0, jnp.int32)])
                d1 = plsc.load_gather(
                    dsl2_v, [rows, jnp.full((16,), l1, jnp.int32)])
                rd0_v[pl.ds(hoff + g * 16, 16)] = 0.5 / (d0 + 1e-16)
                rd1_v[pl.ds(hoff + g * 16, 16)] = 0.5 / (d1 + 1e-16)

        @pl.loop(0, npair)
        def _pair(ip):
            @pl.loop(0, 2)
            def _half(half):
                base = wid * NPW + half * NH
                hoff = half * NH
                pltpu.sync_copy(h_hbm.at[ip].at[pl.ds(base, NH)], dsl_v)
                pltpu.sync_copy(h_hbm.at[npair + ip].at[pl.ds(base, NH)],
                                dsl2_v)

                @pl.loop(0, NH // 16)
                def _nl(g):
                    bv = bid_v[pl.ds(hoff + g * 16, 16)]
                    av = rd0_v[pl.ds(hoff + g * 16, 16)]
                    cv = rd1_v[pl.ds(hoff + g * 16, 16)]
                    mf = jnp.where((base + g * 16 + iot) < N, 0.0,
                                   -jnp.inf).astype(jnp.float32)
                    for l in range(16):
                        j = g * 16 + l
                        bid = bv[l]
                        a = av[l]
                        c = cv[l]
                        mfl = mf[l]
                        for r in range(8):
                            sl = pl.ds(r * 16, 16)
                            osl = pl.ds(ip * 128 + r * 16, 16)
                            x = (dsl_v[j, sl] * a + dsl2_v[j, sl] * c
                                 + bias_v[osl] + mfl)
                            acc_v[bid, osl] = jnp.maximum(acc_v[bid, osl], x)

    pl.run_scoped(work, pltpu.VMEM((NPW // 2, 128), jnp.float32),
                  pltpu.VMEM((NPW // 2, 128), jnp.float32))

    pltpu.sync_copy(acc_v, stage.at[s])
    plsc.subcore_barrier()

    def combine(cmb_v):
        @pl.when(s < 8)
        def _comb_band():
            for t in range(NSUB):
                pltpu.sync_copy(stage.at[t, pl.ds(s * 8, 8)], cmb_v.at[t])

            @pl.loop(0, 8)
            def _comb(r):
                for r2 in range(W // 16):
                    sl = pl.ds(r2 * 16, 16)
                    m = cmb_v[0, r, sl]
                    for t in range(1, NSUB):
                        m = jnp.maximum(m, cmb_v[t, r, sl])
                    acc_v[r, sl] = m

            pltpu.sync_copy(acc_v.at[pl.ds(0, 8)],
                            out_hbm.at[core].at[pl.ds(s * 8, 8)])

    pl.run_scoped(combine, pltpu.VMEM((NSUB, 8, W), jnp.float32))


def sc_gmp(h3d, batch, bias, pairs, dpos):
    NW = 32
    NPW = NP // NW
    W = len(pairs) * 128
    body = functools.partial(_gmp_body, tuple(pairs), tuple(dpos), NW, NPW)
    f = pl.kernel(
        body,
        out_type=jax.ShapeDtypeStruct((2, B, W), jnp.float32),
        mesh=_mesh(),
        compiler_params=_sc_params(),
        scratch_types=[
            pltpu.VMEM((NPW,), jnp.int32),
            pltpu.VMEM((W,), jnp.float32),
            pltpu.VMEM((NPW,), jnp.float32),
            pltpu.VMEM((NPW,), jnp.float32),
            pltpu.VMEM((B, W), jnp.float32),
            pltpu.VMEM_SHARED((NSUB, B, W), jnp.float32),
            pltpu.SemaphoreType.DMA,
        ],
    )
    return f(h3d, batch, bias)


def _dinv_body(p_ref, o_ref):
    d = p_ref[0] + p_ref[1]
    o_ref[...] = jnp.where(d > 0, lax.rsqrt(d), 0.0)


def tc_dinv(partials):
    return pl.pallas_call(
        _dinv_body,
        out_shape=jax.ShapeDtypeStruct((NP,), jnp.float32),
    )(partials)


def _head_body(fouts, x_ref, xt_ref, xq_ref, *refs):
    (mf1w, mf1b, mf2w, mf2b, pf1w, pf1b, pf2w, pf2b, cf1w, cf1b, cf2w, cf2b,
     ax1, axb, ax2, at1, atb, at2, aq1, aqb, aq2,
     f1w, f1b, f2w, f2b, ow, ob, o_ref) = refs
    dot = lambda a, b: jnp.dot(a, b, preferred_element_type=jnp.float32)
    relu = lambda v: jnp.maximum(v, 0.0)

    def pool(ref, fout):
        m = jnp.maximum(ref[0], ref[1])[:, :fout]
        return jnp.where(jnp.isfinite(m), m, 0.0)

    x = pool(x_ref, fouts[0])
    xt = pool(xt_ref, fouts[1])
    xq = pool(xq_ref, fouts[2])
    x = dot(relu(dot(x, mf1w[...]) + mf1b[...]), mf2w[...]) + mf2b[...]
    xt = dot(relu(dot(xt, pf1w[...]) + pf1b[...]), pf2w[...]) + pf2b[...]
    xq = dot(relu(dot(xq, cf1w[...]) + cf1b[...]), cf2w[...]) + cf2b[...]
    ax = dot(jnp.tanh(dot(x, ax1[...]) + axb[...]), ax2[...])
    at = dot(jnp.tanh(dot(xt, at1[...]) + atb[...]), at2[...])
    aq = dot(jnp.tanh(dot(xq, aq1[...]) + aqb[...]), aq2[...])
    a = jnp.concatenate([ax, at, aq], axis=1)
    a = jax.nn.softmax(a, axis=1)
    emb = jnp.concatenate(
        [a[:, 0:1] * x, a[:, 1:2] * xt, a[:, 2:3] * xq], axis=1)
    xc = relu(dot(emb, f1w[...]) + f1b[...])
    xc = relu(dot(xc, f2w[...]) + f2b[...])
    o_ref[...] = dot(xc, ow[...]) + ob[...]


def tc_head(x, xt, xq, params, fouts):
    args = [x, xt, xq]
    for nm in ("mol_fc1", "mol_fc2", "pro_fc1", "pro_fc2", "clq_fc1", "clq_fc2"):
        args += [params[nm]["W"], params[nm]["b"].reshape(1, -1)]
    for nm in ("att_x", "att_xt", "att_xq"):
        args += [params[nm]["W1"], params[nm]["b1"].reshape(1, -1), params[nm]["W2"]]
    for nm in ("fc1", "fc2", "out"):
        args += [params[nm]["W"], params[nm]["b"].reshape(1, -1)]
    return pl.pallas_call(
        functools.partial(_head_body, tuple(fouts)),
        out_shape=jax.ShapeDtypeStruct((B, 1), jnp.float32),
    )(*args)


def _gat_layout(fout):
    hp = ((fout + 1 + 127) // 128) * 128
    npc = hp // 128
    C = 2 * npc
    chunks = [(i, 0) for i in range(npc)] + [(npc + i, 1) for i in range(npc)]
    dch = fout // 128
    dlane = fout - dch * 128
    return dict(fout=fout, hp=hp, npc=npc, C=C, chunks=chunks,
                dpos=((dch, dlane), (npc + dch, dlane)),
                pairs=[(i, npc + i) for i in range(npc)])


def _pack_gat(prm, lay):
    fout, hp, npc = lay["fout"], lay["hp"], lay["npc"]
    Wf = prm["W"]
    fin = Wf.shape[0]
    W0, W1 = Wf[:, :fout], Wf[:, fout:]
    z = jnp.zeros((fin, hp - fout), jnp.float32)
    w = jnp.concatenate([W0, z, W1, z], axis=1)
    ones = jnp.zeros((2 * hp,), jnp.float32)
    ones = ones.at[fout].set(1.0).at[hp + fout].set(1.0).reshape(1, -1)
    wa = jnp.concatenate([
        (W0 @ prm["asrc"][0])[:, None], (W1 @ prm["asrc"][1])[:, None],
        (W0 @ prm["adst"][0])[:, None], (W1 @ prm["adst"][1])[:, None]],
        axis=1)
    bias = jnp.concatenate([prm["b"], jnp.zeros((hp - fout,), jnp.float32)])
    return w, ones, wa, bias


def _pad_rows(w, rows):
    fin = w.shape[0]
    if fin == rows:
        return w
    return jnp.concatenate(
        [w, jnp.zeros((rows - fin,) + w.shape[1:], jnp.float32)], axis=0)


def _edges3(edge_index, extra_w=None):
    src = jnp.concatenate([edge_index[0], jnp.arange(N, dtype=jnp.int32),
                           jnp.zeros((EPAD - EP,), jnp.int32)]).astype(jnp.int32)
    dst = jnp.concatenate([edge_index[1], jnp.arange(N, dtype=jnp.int32),
                           jnp.zeros((EPAD - EP,), jnp.int32)]).astype(jnp.int32)
    val = jnp.concatenate([jnp.ones((EP,), jnp.float32),
                           jnp.zeros((EPAD - EP,), jnp.float32)])
    if extra_w is not None:
        val = jnp.concatenate([extra_w, jnp.ones((N,), jnp.float32),
                               jnp.zeros((EPAD - EP,), jnp.float32)])
    r = lambda a: a.reshape(NSUB, NB, KB)
    return r(src), r(dst), r(val)


def _gat_layer(x_or_u, prm, lay_in, lay, src3, dst3, val3, mode):
    w, ones, wa, bias = _pack_gat(prm, lay)
    if mode == "in2d":
        wk = w
        wak = wa
        bk = jnp.zeros((1, 64), jnp.float32)
    else:
        rows = lay_in["npc"] * 128 if lay_in is not None else None
        wk = _pad_rows(w, rows)
        wak = _pad_rows(wa, rows)
        bk = lay_in["bias"].reshape(1, -1)
    h3d, st = tc_project(x_or_u, wk, bk, ones, wak, mode,
                         dpos=lay_in["dpos"] if mode == "in3d_gat" else None,
                         want_st=True)
    u = sc_spmm(h3d, st.reshape(-1), src3, dst3, val3, lay["chunks"], "gat")
    return u


def kernel(mol_x, pro_x, clique_x, pro_edge_weight, params, mol_edge_index,
           pro_edge_index, clique_edge_index, mol_batch, pro_batch,
           clique_batch):
    mol_edge_index = mol_edge_index.astype(jnp.int32)
    pro_edge_index = pro_edge_index.astype(jnp.int32)
    clique_edge_index = clique_edge_index.astype(jnp.int32)
    padb = jnp.zeros((NP - N,), jnp.int32)
    mol_batch = jnp.concatenate([mol_batch.astype(jnp.int32), padb])
    pro_batch = jnp.concatenate([pro_batch.astype(jnp.int32), padb])
    clique_batch = jnp.concatenate([clique_batch.astype(jnp.int32), padb])

    lay_m = _gat_layout(312)
    lay_p = _gat_layout(132)
    lay_q = _gat_layout(368)

    s3, d3, v3 = _edges3(mol_edge_index)
    u = _gat_layer(mol_x, params["mol0"], None, lay_m, s3, d3, v3, "in2d")
    u = _gat_layer(u, params["mol1"],
                   dict(lay_m, bias=_pack_gat(params["mol0"], lay_m)[3]),
                   lay_m, s3, d3, v3, "in3d_gat")
    u = _gat_layer(u, params["mol2"],
                   dict(lay_m, bias=_pack_gat(params["mol1"], lay_m)[3]),
                   lay_m, s3, d3, v3, "in3d_gat")
    bias_last = jnp.concatenate(
        [params["mol2"]["b"],
         jnp.zeros((lay_m["hp"] - 312,), jnp.float32)])
    xp = sc_gmp(u, mol_batch, bias_last, lay_m["pairs"], lay_m["dpos"])

    s3p, d3p, v3p = _edges3(pro_edge_index)
    _, _, w3p = _edges3(pro_edge_index, extra_w=pro_edge_weight)
    partials = sc_deg(d3p, w3p)
    dinv = tc_dinv(partials[:, :, 0])
    hp_g = ((132 + 127) // 128) * 128
    Wg = jnp.concatenate(
        [params["pro0"]["W"],
         jnp.zeros((33, hp_g - 132), jnp.float32)], axis=1)
    zeros_ones = jnp.zeros((1, hp_g), jnp.float32)
    dummy_wa = jnp.zeros((33, 4), jnp.float32)
    hg = tc_project(pro_x, Wg, jnp.zeros((1, 64), jnp.float32), zeros_ones,
                    dummy_wa, "in2d", want_st=False)
    gchunks = [(i, 0) for i in range(hp_g // 128)]
    ug = sc_spmm(hg, dinv, s3p, d3p, w3p, gchunks, "gcn")
    bias_g = jnp.concatenate(
        [params["pro0"]["b"], jnp.zeros((hp_g - 132,), jnp.float32)])
    w1, ones1, wa1, bias1 = _pack_gat(params["pro1"], lay_p)
    h3d, st = tc_project(ug, _pad_rows(w1, hp_g), bias_g.reshape(1, -1),
                         ones1, _pad_rows(wa1, hp_g), "in3d_gcn",
                         want_st=True)
    u = sc_spmm(h3d, st.reshape(-1), s3p, d3p, v3p, lay_p["chunks"], "gat")
    u = _gat_layer(u, params["pro2"], dict(lay_p, bias=bias1), lay_p,
                   s3p, d3p, v3p, "in3d_gat")
    bias_last = jnp.concatenate(
        [params["pro2"]["b"], jnp.zeros((lay_p["hp"] - 132,), jnp.float32)])
    xtp = sc_gmp(u, pro_batch, bias_last, lay_p["pairs"], lay_p["dpos"])

    s3c, d3c, v3c = _edges3(clique_edge_index)
    u = _gat_layer(clique_x, params["clq0"], None, lay_q, s3c, d3c, v3c, "in2d")
    u = _gat_layer(u, params["clq1"],
                   dict(lay_q, bias=_pack_gat(params["clq0"], lay_q)[3]),
                   lay_q, s3c, d3c, v3c, "in3d_gat")
    u = _gat_layer(u, params["clq2"],
                   dict(lay_q, bias=_pack_gat(params["clq1"], lay_q)[3]),
                   lay_q, s3c, d3c, v3c, "in3d_gat")
    bias_last = jnp.concatenate(
        [params["clq2"]["b"], jnp.zeros((lay_q["hp"] - 368,), jnp.float32)])
    xqp = sc_gmp(u, clique_batch, bias_last, lay_q["pairs"], lay_q["dpos"])

    return tc_head(xp, xtp, xqp, params, (312, 132, 368))
